# baseline (speedup 1.0000x reference)
import jax
import jax.numpy as jnp
from jax import lax
from jax.experimental import pallas as pl
from jax.experimental.pallas import tpu as pltpu
from jax.experimental.pallas import tpu_sc as plsc

B = 16384
D = 16
NC = 2
NS = 16
NW = NC * NS
BPW = B // NW
L = 16
NG = BPW // L
BLK = 128


def _sc_body(bu_hbm, bp_hbm, bn_hbm, ut_hbm, it_hbm,
             outp_hbm, outn_hbm,
             vidx_u, vidx_p, vidx_n,
             blk_u, blk_i, blk_j,
             accp, accn, sem_a, sem_b):
  wid = lax.axis_index("s") * NC + lax.axis_index("c")

  pltpu.sync_copy(bu_hbm.at[wid], vidx_u)
  pltpu.sync_copy(bp_hbm.at[wid], vidx_p)
  pltpu.sync_copy(bn_hbm.at[wid], vidx_n)

  lane = lax.iota(jnp.int32, L)
  sems = (sem_a, sem_b)

  def starts_of(g):
    row = g // 8
    col0 = (g % 8) * L
    vu = vidx_u[row, pl.ds(col0, L)]
    vp = vidx_p[row, pl.ds(col0, L)]
    vn = vidx_n[row, pl.ds(col0, L)]
    return (vu, vp, vn,
            lax.shift_right_logical(vu, 7) * BLK,
            lax.shift_right_logical(vp, 7) * BLK,
            lax.shift_right_logical(vn, 7) * BLK)

  def unit_copies(su, sp, sn, cg):
    fs = pl.ds(cg * 8, 8)
    copies = []
    for r in range(L):
      for starts, tbl, dst in ((su, ut_hbm, blk_u),
                               (sp, it_hbm, blk_i),
                               (sn, it_hbm, blk_j)):
        start = pl.multiple_of(starts[r], BLK)
        copies.append(pltpu.make_async_copy(
            tbl.at[fs, pl.ds(start, BLK)], dst.at[cg, r], sems[cg]))
    return copies

  def partial(cg, ru, rp, rn, ap, an):
    for c in range(8):
      cv = jnp.full((L,), c, jnp.int32)
      uu = plsc.load_gather(blk_u.at[cg], [lane, cv, ru])
      ii = plsc.load_gather(blk_i.at[cg], [lane, cv, rp])
      jj = plsc.load_gather(blk_j.at[cg], [lane, cv, rn])
      ap = ap + uu * ii
      an = an + uu * jj
    return ap, an

  su0, sp0, sn0 = starts_of(0)[3:]
  for cp in unit_copies(su0, sp0, sn0, 0):
    cp.start()

  def group(g, carry):
    vu, vp, vn, su, sp, sn = starts_of(g)
    ru = vu & (BLK - 1)
    rp = vp & (BLK - 1)
    rn = vn & (BLK - 1)

    for cp in unit_copies(su, sp, sn, 1):
      cp.start()
    for cp in unit_copies(su, sp, sn, 0):
      cp.wait()
    ap = jnp.zeros((L,), jnp.float32)
    an = jnp.zeros((L,), jnp.float32)
    ap, an = partial(0, ru, rp, rn, ap, an)

    gn = jnp.minimum(g + 1, NG - 1)
    sun, spn, snn = starts_of(gn)[3:]
    for cp in unit_copies(sun, spn, snn, 0):
      cp.start()
    for cp in unit_copies(su, sp, sn, 1):
      cp.wait()
    ap, an = partial(1, ru, rp, rn, ap, an)

    base = pl.multiple_of(g * L, L)
    accp[pl.ds(base, L)] = ap
    accn[pl.ds(base, L)] = an
    return carry

  lax.fori_loop(0, NG, group, 0)

  suz, spz, snz = starts_of(NG - 1)[3:]
  for cp in unit_copies(suz, spz, snz, 0):
    cp.wait()

  out = pl.ds(wid * BPW, BPW)
  pltpu.sync_copy(accp, outp_hbm.at[out])
  pltpu.sync_copy(accn, outn_hbm.at[out])


@jax.jit
def kernel(batch_user, batch_pos_item, batch_neg_item, user_table, item_table):
  bu = batch_user.reshape(NW, 4, 128)
  bp = batch_pos_item.reshape(NW, 4, 128)
  bn = batch_neg_item.reshape(NW, 4, 128)
  ut = user_table.T
  it = item_table.T

  mesh = plsc.VectorSubcoreMesh(core_axis_name="c", subcore_axis_name="s",
                                num_cores=NC, num_subcores=NS)
  run = pl.kernel(
      _sc_body,
      out_type=(jax.ShapeDtypeStruct((B,), jnp.float32),
                jax.ShapeDtypeStruct((B,), jnp.float32)),
      mesh=mesh,
      scratch_types=[
          pltpu.VMEM((4, 128), jnp.int32),
          pltpu.VMEM((4, 128), jnp.int32),
          pltpu.VMEM((4, 128), jnp.int32),
          pltpu.VMEM((2, L, 8, BLK), jnp.float32),
          pltpu.VMEM((2, L, 8, BLK), jnp.float32),
          pltpu.VMEM((2, L, 8, BLK), jnp.float32),
          pltpu.VMEM((BPW,), jnp.float32),
          pltpu.VMEM((BPW,), jnp.float32),
          pltpu.SemaphoreType.DMA,
          pltpu.SemaphoreType.DMA,
      ],
      compiler_params=pltpu.CompilerParams(needs_layout_passes=False,
                                           disable_bounds_checks=True),
  )
  pos, neg = run(bu, bp, bn, ut, it)
  return (pos.reshape(B, 1), neg.reshape(B, 1))


# speedup vs baseline: 1.0288x; 1.0288x over previous
import jax
import jax.numpy as jnp
from jax import lax
from jax.experimental import pallas as pl
from jax.experimental.pallas import tpu as pltpu
from jax.experimental.pallas import tpu_sc as plsc

B = 16384
D = 16
NC = 2
NS = 16
NW = NC * NS
BPW = B // NW
L = 16
NG = BPW // L
BLK = 128


def _sc_body(bu_hbm, bp_hbm, bn_hbm, ut_hbm, it_hbm,
             outp_hbm, outn_hbm,
             vidx_u, vidx_p, vidx_n,
             blk_u, blk_i, blk_j,
             accp, accn, sem):
  wid = lax.axis_index("s") * NC + lax.axis_index("c")

  pltpu.sync_copy(bu_hbm.at[wid], vidx_u)
  pltpu.sync_copy(bp_hbm.at[wid], vidx_p)
  pltpu.sync_copy(bn_hbm.at[wid], vidx_n)

  lane = lax.iota(jnp.int32, L)

  def group(g, carry):
    row = g // 8
    col0 = (g % 8) * L
    vu = vidx_u[row, pl.ds(col0, L)]
    vp = vidx_p[row, pl.ds(col0, L)]
    vn = vidx_n[row, pl.ds(col0, L)]
    su = lax.shift_right_logical(vu, 7) * BLK
    sp = lax.shift_right_logical(vp, 7) * BLK
    sn = lax.shift_right_logical(vn, 7) * BLK
    copies = []
    for r in range(L):
      for starts, tbl, dst in ((su, ut_hbm, blk_u),
                               (sp, it_hbm, blk_i),
                               (sn, it_hbm, blk_j)):
        start = pl.multiple_of(starts[r], BLK)
        copies.append(pltpu.make_async_copy(
            tbl.at[:, pl.ds(start, BLK)], dst.at[r], sem))
    for cp in copies:
      cp.start()
    for cp in copies:
      cp.wait()

    ru = vu & (BLK - 1)
    rp = vp & (BLK - 1)
    rn = vn & (BLK - 1)
    ap = jnp.zeros((L,), jnp.float32)
    an = jnp.zeros((L,), jnp.float32)
    for c in range(D):
      cv = jnp.full((L,), c, jnp.int32)
      uu = plsc.load_gather(blk_u, [lane, cv, ru])
      ii = plsc.load_gather(blk_i, [lane, cv, rp])
      jj = plsc.load_gather(blk_j, [lane, cv, rn])
      ap = ap + uu * ii
      an = an + uu * jj
    base = pl.multiple_of(g * L, L)
    accp[pl.ds(base, L)] = ap
    accn[pl.ds(base, L)] = an
    return carry

  lax.fori_loop(0, NG, group, 0)

  out = pl.ds(wid * BPW, BPW)
  pltpu.sync_copy(accp, outp_hbm.at[out])
  pltpu.sync_copy(accn, outn_hbm.at[out])


@jax.jit
def kernel(batch_user, batch_pos_item, batch_neg_item, user_table, item_table):
  bu = batch_user.reshape(NW, 4, 128)
  bp = batch_pos_item.reshape(NW, 4, 128)
  bn = batch_neg_item.reshape(NW, 4, 128)
  ut = user_table.T
  it = item_table.T

  mesh = plsc.VectorSubcoreMesh(core_axis_name="c", subcore_axis_name="s",
                                num_cores=NC, num_subcores=NS)
  run = pl.kernel(
      _sc_body,
      out_type=(jax.ShapeDtypeStruct((B,), jnp.float32),
                jax.ShapeDtypeStruct((B,), jnp.float32)),
      mesh=mesh,
      scratch_types=[
          pltpu.VMEM((4, 128), jnp.int32),
          pltpu.VMEM((4, 128), jnp.int32),
          pltpu.VMEM((4, 128), jnp.int32),
          pltpu.VMEM((L, D, BLK), jnp.float32),
          pltpu.VMEM((L, D, BLK), jnp.float32),
          pltpu.VMEM((L, D, BLK), jnp.float32),
          pltpu.VMEM((BPW,), jnp.float32),
          pltpu.VMEM((BPW,), jnp.float32),
          pltpu.SemaphoreType.DMA,
      ],
      compiler_params=pltpu.CompilerParams(needs_layout_passes=False,
                                           disable_bounds_checks=True),
  )
  pos, neg = run(bu, bp, bn, ut, it)
  return (pos.reshape(B, 1), neg.reshape(B, 1))
